# Initial kernel scaffold; baseline (speedup 1.0000x reference)
import jax
import jax.numpy as jnp
from jax import lax
from jax.experimental import pallas as pl
from jax.experimental.pallas import tpu as pltpu
from jax.experimental.pallas import tpu_sc as plsc

NUM_HEADS = 32
EDGE_HIDDEN = 32
MULTI_HOP_MAX_DIST = 5
NUM_EDGES = 1536
N_GRAPH = 16
N_NODE = 64

_NPAIR = N_GRAPH * N_NODE * N_NODE
_TBL_ROWS = NUM_EDGES + 1

_info = plsc.get_sparse_core_info()
_NC, _NS = _info.num_cores, _info.num_subcores
_NW = _NC * _NS
_CHUNK = _NPAIR // _NW
_GROUPS = _CHUNK // 16


def _tpre_body(e_ref, w_ref, o_ref):
    o_ref[0] = jnp.dot(e_ref[...], w_ref[0], preferred_element_type=jnp.float32)


def _tc_pre(edge_enc_w, w):
    return pl.pallas_call(
        _tpre_body,
        grid=(MULTI_HOP_MAX_DIST,),
        in_specs=[
            pl.BlockSpec((_TBL_ROWS, EDGE_HIDDEN), lambda d: (0, 0)),
            pl.BlockSpec((1, EDGE_HIDDEN, NUM_HEADS), lambda d: (d, 0, 0)),
        ],
        out_specs=pl.BlockSpec((1, _TBL_ROWS, NUM_HEADS), lambda d: (d, 0, 0)),
        out_shape=jax.ShapeDtypeStruct(
            (MULTI_HOP_MAX_DIST, _TBL_ROWS, NUM_HEADS), jnp.float32),
    )(edge_enc_w, w)


def _sc_body(t5_hbm, spw_hbm, idx_hbm, spos_hbm, out_hbm, tbl, acc, idx3, spos):
    wid = lax.axis_index("s") * _NC + lax.axis_index("c")
    base = wid * _CHUNK

    for dd in range(MULTI_HOP_MAX_DIST):
        pltpu.sync_copy(t5_hbm.at[dd], tbl)
        pltpu.sync_copy(idx_hbm.at[dd, :, pl.ds(base, _CHUNK)], idx3)

        def eround(g, carry, first=(dd == 0)):
            p0 = pl.multiple_of(g * 16, 16)
            i0 = idx3[0, pl.ds(p0, 16)]
            i1 = idx3[1, pl.ds(p0, 16)]
            i2 = idx3[2, pl.ds(p0, 16)]
            for c in range(NUM_HEADS):
                cc = jnp.full((16,), c, jnp.int32)
                v = (plsc.load_gather(tbl, [i0, cc])
                     + plsc.load_gather(tbl, [i1, cc])
                     + plsc.load_gather(tbl, [i2, cc]))
                if first:
                    acc[c, pl.ds(p0, 16)] = v
                else:
                    plsc.addupdate(acc.at[c, pl.ds(p0, 16)], v)
            return carry

        lax.fori_loop(0, _GROUPS, eround, 0)

    pltpu.sync_copy(spw_hbm, tbl)
    pltpu.sync_copy(spos_hbm.at[pl.ds(base, _CHUNK)], spos)

    def sround(g, carry):
        p0 = pl.multiple_of(g * 16, 16)
        sv = spos[pl.ds(p0, 16)]
        sp = jnp.where(sv == 0, 1, sv)
        sp = jnp.where(sp > 1, sp - 1, sp)
        sp = jnp.minimum(sp, MULTI_HOP_MAX_DIST)
        recip = (1.0 / 3.0) / sp.astype(jnp.float32)
        for c in range(NUM_HEADS):
            cc = jnp.full((16,), c, jnp.int32)
            v = acc[c, pl.ds(p0, 16)] * recip + plsc.load_gather(tbl, [sv, cc])
            acc[c, pl.ds(p0, 16)] = v
        return carry

    lax.fori_loop(0, _GROUPS, sround, 0)

    pltpu.sync_copy(acc, out_hbm.at[:, pl.ds(base, _CHUNK)])


_sc_kernel = pl.kernel(
    _sc_body,
    out_type=jax.ShapeDtypeStruct((NUM_HEADS, _NPAIR), jnp.float32),
    mesh=plsc.VectorSubcoreMesh(core_axis_name="c", subcore_axis_name="s"),
    scratch_types=[
        pltpu.VMEM((_TBL_ROWS, NUM_HEADS), jnp.float32),
        pltpu.VMEM((NUM_HEADS, _CHUNK), jnp.float32),
        pltpu.VMEM((3, _CHUNK), jnp.int32),
        pltpu.VMEM((_CHUNK,), jnp.int32),
    ],
)


def _asm_body(ab_ref, rt_ref, t_ref, o_ref):
    ab2 = ab_ref[0] * 2.0
    inner = rt_ref[:, 0]
    t = t_ref[0]
    n1 = N_NODE + 1
    ii = lax.broadcasted_iota(jnp.int32, (NUM_HEADS, n1, n1), 1)
    jj = lax.broadcasted_iota(jnp.int32, (NUM_HEADS, n1, n1), 2)
    border = (ii == 0) | (jj == 0)
    tb = jnp.where(border,
                   jnp.broadcast_to(t[:, None, None], (NUM_HEADS, n1, n1)),
                   jnp.zeros((NUM_HEADS, n1, n1), jnp.float32))
    z_col = jnp.zeros((NUM_HEADS, N_NODE, 1), jnp.float32)
    z_row = jnp.zeros((NUM_HEADS, 1, n1), jnp.float32)
    padded = jnp.concatenate(
        [z_row, jnp.concatenate([z_col, inner], axis=2)], axis=1)
    o_ref[0] = ab2[None] + tb + padded


def _tc_asm(attn_bias, rt4, gtvd_w):
    n1 = N_NODE + 1
    return pl.pallas_call(
        _asm_body,
        grid=(N_GRAPH,),
        in_specs=[
            pl.BlockSpec((1, n1, n1), lambda g: (g, 0, 0)),
            pl.BlockSpec((NUM_HEADS, 1, N_NODE, N_NODE), lambda g: (0, g, 0, 0)),
            pl.BlockSpec((1, NUM_HEADS), lambda g: (0, 0)),
        ],
        out_specs=pl.BlockSpec((1, NUM_HEADS, n1, n1), lambda g: (g, 0, 0, 0)),
        out_shape=jax.ShapeDtypeStruct(
            (N_GRAPH, NUM_HEADS, n1, n1), jnp.float32),
    )(attn_bias, rt4, gtvd_w)


def kernel(attn_bias, node_attr, is_molecule, spatial_pos, edge_input,
           spatial_pos_w, gtvd_w, edge_enc_w, edge_dis_w):
    w = edge_dis_w.reshape(-1, EDGE_HIDDEN, NUM_HEADS)[:MULTI_HOP_MAX_DIST]
    t5 = _tc_pre(edge_enc_w, w)

    spw_pad = jnp.zeros((_TBL_ROWS, NUM_HEADS), jnp.float32)
    spw_pad = spw_pad.at[:spatial_pos_w.shape[0]].set(spatial_pos_w)
    idx_t = jnp.transpose(
        edge_input.reshape(_NPAIR, MULTI_HOP_MAX_DIST, 3), (1, 2, 0))
    spos_flat = spatial_pos.reshape(_NPAIR)

    rt = _sc_kernel(t5, spw_pad, idx_t, spos_flat)
    rt4 = rt.reshape(NUM_HEADS, N_GRAPH, N_NODE, N_NODE)
    return _tc_asm(attn_bias, rt4, gtvd_w)


# speedup vs baseline: 6.6935x; 6.6935x over previous
import jax
import jax.numpy as jnp
from jax import lax
from jax.experimental import pallas as pl
from jax.experimental.pallas import tpu as pltpu
from jax.experimental.pallas import tpu_sc as plsc

NUM_HEADS = 32
EDGE_HIDDEN = 32
MULTI_HOP_MAX_DIST = 5
NUM_EDGES = 1536
N_GRAPH = 16
N_NODE = 64

_NPAIR = N_GRAPH * N_NODE * N_NODE
_TBL_ROWS = NUM_EDGES + 1
_TROWS_P = 1544
_TSIZE = _TROWS_P * NUM_HEADS

_info = plsc.get_sparse_core_info()
_NC, _NS = _info.num_cores, _info.num_subcores
_NW = _NC * _NS
_CHUNK = _NPAIR // _NW
_GROUPS = _CHUNK // 16


def _tpre_body(e_ref, w_ref, o_ref):
    o_ref[0] = jnp.dot(e_ref[...], w_ref[0], preferred_element_type=jnp.float32)


def _tc_pre(edge_enc_w_pad, w):
    return pl.pallas_call(
        _tpre_body,
        grid=(MULTI_HOP_MAX_DIST,),
        in_specs=[
            pl.BlockSpec((_TROWS_P, EDGE_HIDDEN), lambda d: (0, 0)),
            pl.BlockSpec((1, EDGE_HIDDEN, NUM_HEADS), lambda d: (d, 0, 0)),
        ],
        out_specs=pl.BlockSpec((1, _TROWS_P, NUM_HEADS), lambda d: (d, 0, 0)),
        out_shape=jax.ShapeDtypeStruct(
            (MULTI_HOP_MAX_DIST, _TROWS_P, NUM_HEADS), jnp.float32),
    )(edge_enc_w_pad, w)


def _sc_body(t5_hbm, spw_hbm, idx_hbm, spos_hbm, out_hbm, tbl, acc, idx3, spos):
    wid = lax.axis_index("s") * _NC + lax.axis_index("c")
    base = wid * _CHUNK

    for dd in range(MULTI_HOP_MAX_DIST):
        pltpu.sync_copy(t5_hbm.at[pl.ds(dd * _TSIZE, _TSIZE)], tbl)
        for k in range(3):
            pltpu.sync_copy(
                idx_hbm.at[pl.ds((dd * 3 + k) * _NPAIR + base, _CHUNK)],
                idx3.at[pl.ds(k * _CHUNK, _CHUNK)])

        def eround(g, carry, first=(dd == 0)):
            p0 = pl.multiple_of(g * 16, 16)
            i0 = idx3[pl.ds(p0, 16)] * NUM_HEADS
            i1 = idx3[pl.ds(_CHUNK + p0, 16)] * NUM_HEADS
            i2 = idx3[pl.ds(2 * _CHUNK + p0, 16)] * NUM_HEADS
            for c in range(NUM_HEADS):
                cc = jnp.full((16,), c, jnp.int32)
                v = (plsc.load_gather(tbl, [i0 + cc])
                     + plsc.load_gather(tbl, [i1 + cc])
                     + plsc.load_gather(tbl, [i2 + cc]))
                if first:
                    acc[c, pl.ds(p0, 16)] = v
                else:
                    plsc.addupdate(acc.at[c, pl.ds(p0, 16)], v)
            return carry

        lax.fori_loop(0, _GROUPS, eround, 0)

    pltpu.sync_copy(spw_hbm, tbl)
    pltpu.sync_copy(spos_hbm.at[pl.ds(base, _CHUNK)], spos)

    def sround(g, carry):
        p0 = pl.multiple_of(g * 16, 16)
        sv = spos[pl.ds(p0, 16)]
        sp = jnp.where(sv == 0, 1, sv)
        sp = jnp.where(sp > 1, sp - 1, sp)
        sp = jnp.minimum(sp, MULTI_HOP_MAX_DIST)
        recip = (1.0 / 3.0) / sp.astype(jnp.float32)
        svs = sv * NUM_HEADS
        for c in range(NUM_HEADS):
            cc = jnp.full((16,), c, jnp.int32)
            v = (acc[c, pl.ds(p0, 16)] * recip
                 + plsc.load_gather(tbl, [svs + cc]))
            acc[c, pl.ds(p0, 16)] = v
        return carry

    lax.fori_loop(0, _GROUPS, sround, 0)

    pltpu.sync_copy(acc, out_hbm.at[:, pl.ds(base, _CHUNK)])


_sc_kernel = pl.kernel(
    _sc_body,
    out_type=jax.ShapeDtypeStruct((NUM_HEADS, _NPAIR), jnp.float32),
    mesh=plsc.VectorSubcoreMesh(core_axis_name="c", subcore_axis_name="s"),
    compiler_params=pltpu.CompilerParams(needs_layout_passes=False),
    scratch_types=[
        pltpu.VMEM((_TSIZE,), jnp.float32),
        pltpu.VMEM((NUM_HEADS, _CHUNK), jnp.float32),
        pltpu.VMEM((3 * _CHUNK,), jnp.int32),
        pltpu.VMEM((_CHUNK,), jnp.int32),
    ],
)


def _asm_body(ab_ref, rt_ref, t_ref, o_ref):
    ab2 = ab_ref[0] * 2.0
    inner = rt_ref[:, 0]
    t = t_ref[0]
    n1 = N_NODE + 1
    ii = lax.broadcasted_iota(jnp.int32, (NUM_HEADS, n1, n1), 1)
    jj = lax.broadcasted_iota(jnp.int32, (NUM_HEADS, n1, n1), 2)
    border = (ii == 0) | (jj == 0)
    tb = jnp.where(border,
                   jnp.broadcast_to(t[:, None, None], (NUM_HEADS, n1, n1)),
                   jnp.zeros((NUM_HEADS, n1, n1), jnp.float32))
    z_col = jnp.zeros((NUM_HEADS, N_NODE, 1), jnp.float32)
    z_row = jnp.zeros((NUM_HEADS, 1, n1), jnp.float32)
    padded = jnp.concatenate(
        [z_row, jnp.concatenate([z_col, inner], axis=2)], axis=1)
    o_ref[0] = ab2[None] + tb + padded


def _tc_asm(attn_bias, rt4, gtvd_w):
    n1 = N_NODE + 1
    return pl.pallas_call(
        _asm_body,
        grid=(N_GRAPH,),
        in_specs=[
            pl.BlockSpec((1, n1, n1), lambda g: (g, 0, 0)),
            pl.BlockSpec((NUM_HEADS, 1, N_NODE, N_NODE), lambda g: (0, g, 0, 0)),
            pl.BlockSpec((1, NUM_HEADS), lambda g: (0, 0)),
        ],
        out_specs=pl.BlockSpec((1, NUM_HEADS, n1, n1), lambda g: (g, 0, 0, 0)),
        out_shape=jax.ShapeDtypeStruct(
            (N_GRAPH, NUM_HEADS, n1, n1), jnp.float32),
    )(attn_bias, rt4, gtvd_w)


def kernel(attn_bias, node_attr, is_molecule, spatial_pos, edge_input,
           spatial_pos_w, gtvd_w, edge_enc_w, edge_dis_w):
    w = edge_dis_w.reshape(-1, EDGE_HIDDEN, NUM_HEADS)[:MULTI_HOP_MAX_DIST]
    eew_pad = jnp.zeros((_TROWS_P, EDGE_HIDDEN), jnp.float32)
    eew_pad = eew_pad.at[:_TBL_ROWS].set(edge_enc_w)
    t5 = _tc_pre(eew_pad, w).reshape(MULTI_HOP_MAX_DIST * _TSIZE)

    spw_pad = jnp.zeros((_TROWS_P, NUM_HEADS), jnp.float32)
    spw_pad = spw_pad.at[:spatial_pos_w.shape[0]].set(spatial_pos_w)
    spw_pad = spw_pad.reshape(_TSIZE)
    idx_t = jnp.transpose(
        edge_input.reshape(_NPAIR, MULTI_HOP_MAX_DIST, 3), (1, 2, 0))
    idx_t = idx_t.reshape(MULTI_HOP_MAX_DIST * 3 * _NPAIR)
    spos_flat = spatial_pos.reshape(_NPAIR)

    rt = _sc_kernel(t5, spw_pad, idx_t, spos_flat)
    rt4 = rt.reshape(NUM_HEADS, N_GRAPH, N_NODE, N_NODE)
    return _tc_asm(attn_bias, rt4, gtvd_w)


# speedup vs baseline: 16.9891x; 2.5381x over previous
import jax
import jax.numpy as jnp
from jax import lax
from jax.experimental import pallas as pl
from jax.experimental.pallas import tpu as pltpu
from jax.experimental.pallas import tpu_sc as plsc

NUM_HEADS = 32
EDGE_HIDDEN = 32
MULTI_HOP_MAX_DIST = 5
NUM_EDGES = 1536
N_GRAPH = 16
N_NODE = 64

_NPAIR = N_GRAPH * N_NODE * N_NODE
_TBL_ROWS = NUM_EDGES + 1
_TROWS_P = 1544
_TSTRIDE = 33
_TSIZE = _TROWS_P * _TSTRIDE

_info = plsc.get_sparse_core_info()
_NC, _NS = _info.num_cores, _info.num_subcores
_NW = _NC * _NS
_CHUNK = _NPAIR // _NW
_GROUPS = _CHUNK // 16


def _tpre_body(e_ref, w_ref, o_ref):
    o_ref[0] = jnp.dot(e_ref[...], w_ref[0], preferred_element_type=jnp.float32)


def _tc_pre(edge_enc_w_pad, w):
    return pl.pallas_call(
        _tpre_body,
        grid=(MULTI_HOP_MAX_DIST,),
        in_specs=[
            pl.BlockSpec((_TROWS_P, EDGE_HIDDEN), lambda d: (0, 0)),
            pl.BlockSpec((1, EDGE_HIDDEN, NUM_HEADS), lambda d: (d, 0, 0)),
        ],
        out_specs=pl.BlockSpec((1, _TROWS_P, NUM_HEADS), lambda d: (d, 0, 0)),
        out_shape=jax.ShapeDtypeStruct(
            (MULTI_HOP_MAX_DIST, _TROWS_P, NUM_HEADS), jnp.float32),
    )(edge_enc_w_pad, w)


def _sc_body(t5_hbm, spw_hbm, idx_hbm, spos_hbm, out_hbm, tbl, acc, idx3, spos):
    wid = lax.axis_index("s") * _NC + lax.axis_index("c")
    base = wid * _CHUNK

    for dd in range(MULTI_HOP_MAX_DIST):
        pltpu.sync_copy(t5_hbm.at[pl.ds(dd * _TSIZE, _TSIZE)], tbl)
        for k in range(3):
            pltpu.sync_copy(
                idx_hbm.at[pl.ds((dd * 3 + k) * _NPAIR + base, _CHUNK)],
                idx3.at[pl.ds(k * _CHUNK, _CHUNK)])

        def eround(g, carry, first=(dd == 0)):
            p0 = pl.multiple_of(g * 16, 16)
            i0 = idx3[pl.ds(p0, 16)] * _TSTRIDE
            i1 = idx3[pl.ds(_CHUNK + p0, 16)] * _TSTRIDE
            i2 = idx3[pl.ds(2 * _CHUNK + p0, 16)] * _TSTRIDE
            for c in range(NUM_HEADS):
                cc = jnp.full((16,), c, jnp.int32)
                v = (plsc.load_gather(tbl, [i0 + cc])
                     + plsc.load_gather(tbl, [i1 + cc])
                     + plsc.load_gather(tbl, [i2 + cc]))
                if first:
                    acc[c, pl.ds(p0, 16)] = v
                else:
                    plsc.addupdate(acc.at[c, pl.ds(p0, 16)], v)
            return carry

        lax.fori_loop(0, _GROUPS, eround, 0)

    pltpu.sync_copy(spw_hbm, tbl)
    pltpu.sync_copy(spos_hbm.at[pl.ds(base, _CHUNK)], spos)

    def sround(g, carry):
        p0 = pl.multiple_of(g * 16, 16)
        sv = spos[pl.ds(p0, 16)]
        sp = jnp.where(sv == 0, 1, sv)
        sp = jnp.where(sp > 1, sp - 1, sp)
        sp = jnp.minimum(sp, MULTI_HOP_MAX_DIST)
        recip = (1.0 / 3.0) / sp.astype(jnp.float32)
        svs = sv * _TSTRIDE
        for c in range(NUM_HEADS):
            cc = jnp.full((16,), c, jnp.int32)
            v = (acc[c, pl.ds(p0, 16)] * recip
                 + plsc.load_gather(tbl, [svs + cc]))
            acc[c, pl.ds(p0, 16)] = v
        return carry

    lax.fori_loop(0, _GROUPS, sround, 0)

    pltpu.sync_copy(acc, out_hbm.at[:, pl.ds(base, _CHUNK)])


_sc_kernel = pl.kernel(
    _sc_body,
    out_type=jax.ShapeDtypeStruct((NUM_HEADS, _NPAIR), jnp.float32),
    mesh=plsc.VectorSubcoreMesh(core_axis_name="c", subcore_axis_name="s"),
    compiler_params=pltpu.CompilerParams(needs_layout_passes=False),
    scratch_types=[
        pltpu.VMEM((_TSIZE,), jnp.float32),
        pltpu.VMEM((NUM_HEADS, _CHUNK), jnp.float32),
        pltpu.VMEM((3 * _CHUNK,), jnp.int32),
        pltpu.VMEM((_CHUNK,), jnp.int32),
    ],
)


def _asm_body(ab_ref, rt_ref, t_ref, o_ref):
    ab2 = ab_ref[0] * 2.0
    inner = rt_ref[:, 0]
    t = t_ref[0]
    n1 = N_NODE + 1
    ii = lax.broadcasted_iota(jnp.int32, (NUM_HEADS, n1, n1), 1)
    jj = lax.broadcasted_iota(jnp.int32, (NUM_HEADS, n1, n1), 2)
    border = (ii == 0) | (jj == 0)
    tb = jnp.where(border,
                   jnp.broadcast_to(t[:, None, None], (NUM_HEADS, n1, n1)),
                   jnp.zeros((NUM_HEADS, n1, n1), jnp.float32))
    z_col = jnp.zeros((NUM_HEADS, N_NODE, 1), jnp.float32)
    z_row = jnp.zeros((NUM_HEADS, 1, n1), jnp.float32)
    padded = jnp.concatenate(
        [z_row, jnp.concatenate([z_col, inner], axis=2)], axis=1)
    o_ref[0] = ab2[None] + tb + padded


def _tc_asm(attn_bias, rt4, gtvd_w):
    n1 = N_NODE + 1
    return pl.pallas_call(
        _asm_body,
        grid=(N_GRAPH,),
        in_specs=[
            pl.BlockSpec((1, n1, n1), lambda g: (g, 0, 0)),
            pl.BlockSpec((NUM_HEADS, 1, N_NODE, N_NODE), lambda g: (0, g, 0, 0)),
            pl.BlockSpec((1, NUM_HEADS), lambda g: (0, 0)),
        ],
        out_specs=pl.BlockSpec((1, NUM_HEADS, n1, n1), lambda g: (g, 0, 0, 0)),
        out_shape=jax.ShapeDtypeStruct(
            (N_GRAPH, NUM_HEADS, n1, n1), jnp.float32),
    )(attn_bias, rt4, gtvd_w)


def kernel(attn_bias, node_attr, is_molecule, spatial_pos, edge_input,
           spatial_pos_w, gtvd_w, edge_enc_w, edge_dis_w):
    w = edge_dis_w.reshape(-1, EDGE_HIDDEN, NUM_HEADS)[:MULTI_HOP_MAX_DIST]
    eew_pad = jnp.zeros((_TROWS_P, EDGE_HIDDEN), jnp.float32)
    eew_pad = eew_pad.at[:_TBL_ROWS].set(edge_enc_w)
    t5 = _tc_pre(eew_pad, w)
    t5 = jnp.pad(t5, ((0, 0), (0, 0), (0, _TSTRIDE - NUM_HEADS)))
    t5 = t5.reshape(MULTI_HOP_MAX_DIST * _TSIZE)

    spw_pad = jnp.zeros((_TROWS_P, _TSTRIDE), jnp.float32)
    spw_pad = spw_pad.at[:spatial_pos_w.shape[0], :NUM_HEADS].set(spatial_pos_w)
    spw_pad = spw_pad.reshape(_TSIZE)
    idx_t = jnp.transpose(
        edge_input.reshape(_NPAIR, MULTI_HOP_MAX_DIST, 3), (1, 2, 0))
    idx_t = idx_t.reshape(MULTI_HOP_MAX_DIST * 3 * _NPAIR)
    spos_flat = spatial_pos.reshape(_NPAIR)

    rt = _sc_kernel(t5, spw_pad, idx_t, spos_flat)
    rt4 = rt.reshape(NUM_HEADS, N_GRAPH, N_NODE, N_NODE)
    return _tc_asm(attn_bias, rt4, gtvd_w)


# speedup vs baseline: 25.6016x; 1.5069x over previous
import jax
import jax.numpy as jnp
from jax import lax
from jax.experimental import pallas as pl
from jax.experimental.pallas import tpu as pltpu
from jax.experimental.pallas import tpu_sc as plsc

NUM_HEADS = 32
EDGE_HIDDEN = 32
MULTI_HOP_MAX_DIST = 5
NUM_EDGES = 1536
N_GRAPH = 16
N_NODE = 64

_NPAIR = N_GRAPH * N_NODE * N_NODE
_TBL_ROWS = NUM_EDGES + 1
_TROWS_P = 1544
_TSTRIDE = 17
_TSIZE = _TROWS_P * _TSTRIDE

_info = plsc.get_sparse_core_info()
_NC, _NS = _info.num_cores, _info.num_subcores
_NW = _NC * _NS
_CHUNK = _NPAIR // _NW
_GROUPS = _CHUNK // 16


def _tpre_body(e_ref, w_ref, o_ref):
    o_ref[0] = jnp.dot(e_ref[...], w_ref[0], preferred_element_type=jnp.float32)


def _tc_pre(edge_enc_w_pad, w):
    return pl.pallas_call(
        _tpre_body,
        grid=(MULTI_HOP_MAX_DIST,),
        in_specs=[
            pl.BlockSpec((_TROWS_P, EDGE_HIDDEN), lambda d: (0, 0)),
            pl.BlockSpec((1, EDGE_HIDDEN, NUM_HEADS), lambda d: (d, 0, 0)),
        ],
        out_specs=pl.BlockSpec((1, _TROWS_P, NUM_HEADS), lambda d: (d, 0, 0)),
        out_shape=jax.ShapeDtypeStruct(
            (MULTI_HOP_MAX_DIST, _TROWS_P, NUM_HEADS), jnp.float32),
    )(edge_enc_w_pad, w)


def _gather_pair(tbl, addr):
    w = plsc.load_gather(tbl, [addr])
    bf = plsc.bitcast(w, jnp.bfloat16)
    return plsc.unpack(bf, format=plsc.PackFormat.INTERLEAVED)


def _sc_body(tp_hbm, idx_hbm, spos_hbm, out_hbm, tbl_a, tbl_b, acc, idx6):
    wid = lax.axis_index("s") * _NC + lax.axis_index("c")
    base = wid * _CHUNK

    for rnd, (da, db) in enumerate(((0, 1), (2, 3))):
        pltpu.sync_copy(tp_hbm.at[pl.ds(da * _TSIZE, _TSIZE)], tbl_a)
        pltpu.sync_copy(tp_hbm.at[pl.ds(db * _TSIZE, _TSIZE)], tbl_b)
        for k in range(3):
            pltpu.sync_copy(
                idx_hbm.at[pl.ds((da * 3 + k) * _NPAIR + base, _CHUNK)],
                idx6.at[pl.ds(k * _CHUNK, _CHUNK)])
            pltpu.sync_copy(
                idx_hbm.at[pl.ds((db * 3 + k) * _NPAIR + base, _CHUNK)],
                idx6.at[pl.ds((3 + k) * _CHUNK, _CHUNK)])

        def eround(g, carry, first=(rnd == 0)):
            p0 = pl.multiple_of(g * 16, 16)
            ia = [idx6[pl.ds(k * _CHUNK + p0, 16)] * _TSTRIDE for k in range(3)]
            ib = [idx6[pl.ds((3 + k) * _CHUNK + p0, 16)] * _TSTRIDE
                  for k in range(3)]
            for cp in range(NUM_HEADS // 2):
                cc = jnp.full((16,), cp, jnp.int32)
                lo = None
                hi = None
                for k in range(3):
                    la, ha = _gather_pair(tbl_a, ia[k] + cc)
                    lb, hb = _gather_pair(tbl_b, ib[k] + cc)
                    lo = la + lb if lo is None else lo + la + lb
                    hi = ha + hb if hi is None else hi + ha + hb
                if first:
                    acc[2 * cp, pl.ds(p0, 16)] = lo
                    acc[2 * cp + 1, pl.ds(p0, 16)] = hi
                else:
                    plsc.addupdate(acc.at[2 * cp, pl.ds(p0, 16)], lo)
                    plsc.addupdate(acc.at[2 * cp + 1, pl.ds(p0, 16)], hi)
            return carry

        lax.fori_loop(0, _GROUPS, eround, 0)

    pltpu.sync_copy(tp_hbm.at[pl.ds(4 * _TSIZE, _TSIZE)], tbl_a)
    pltpu.sync_copy(tp_hbm.at[pl.ds(5 * _TSIZE, _TSIZE)], tbl_b)
    for k in range(3):
        pltpu.sync_copy(
            idx_hbm.at[pl.ds((4 * 3 + k) * _NPAIR + base, _CHUNK)],
            idx6.at[pl.ds(k * _CHUNK, _CHUNK)])
    pltpu.sync_copy(spos_hbm.at[pl.ds(base, _CHUNK)],
                    idx6.at[pl.ds(3 * _CHUNK, _CHUNK)])

    def sround(g, carry):
        p0 = pl.multiple_of(g * 16, 16)
        ia = [idx6[pl.ds(k * _CHUNK + p0, 16)] * _TSTRIDE for k in range(3)]
        sv = idx6[pl.ds(3 * _CHUNK + p0, 16)]
        sp = jnp.where(sv == 0, 1, sv)
        sp = jnp.where(sp > 1, sp - 1, sp)
        sp = jnp.minimum(sp, MULTI_HOP_MAX_DIST)
        recip = (1.0 / 3.0) / sp.astype(jnp.float32)
        svs = sv * _TSTRIDE
        for cp in range(NUM_HEADS // 2):
            cc = jnp.full((16,), cp, jnp.int32)
            lo = None
            hi = None
            for k in range(3):
                la, ha = _gather_pair(tbl_a, ia[k] + cc)
                lo = la if lo is None else lo + la
                hi = ha if hi is None else hi + ha
            slo, shi = _gather_pair(tbl_b, svs + cc)
            acc[2 * cp, pl.ds(p0, 16)] = (
                (acc[2 * cp, pl.ds(p0, 16)] + lo) * recip + slo)
            acc[2 * cp + 1, pl.ds(p0, 16)] = (
                (acc[2 * cp + 1, pl.ds(p0, 16)] + hi) * recip + shi)
        return carry

    lax.fori_loop(0, _GROUPS, sround, 0)

    pltpu.sync_copy(acc, out_hbm.at[:, pl.ds(base, _CHUNK)])


_sc_kernel = pl.kernel(
    _sc_body,
    out_type=jax.ShapeDtypeStruct((NUM_HEADS, _NPAIR), jnp.float32),
    mesh=plsc.VectorSubcoreMesh(core_axis_name="c", subcore_axis_name="s"),
    compiler_params=pltpu.CompilerParams(needs_layout_passes=False),
    scratch_types=[
        pltpu.VMEM((_TSIZE,), jnp.int32),
        pltpu.VMEM((_TSIZE,), jnp.int32),
        pltpu.VMEM((NUM_HEADS, _CHUNK), jnp.float32),
        pltpu.VMEM((6 * _CHUNK,), jnp.int32),
    ],
)


def _asm_body(ab_ref, rt_ref, t_ref, o_ref):
    ab2 = ab_ref[0] * 2.0
    inner = rt_ref[:, 0]
    t = t_ref[0]
    n1 = N_NODE + 1
    ii = lax.broadcasted_iota(jnp.int32, (NUM_HEADS, n1, n1), 1)
    jj = lax.broadcasted_iota(jnp.int32, (NUM_HEADS, n1, n1), 2)
    border = (ii == 0) | (jj == 0)
    tb = jnp.where(border,
                   jnp.broadcast_to(t[:, None, None], (NUM_HEADS, n1, n1)),
                   jnp.zeros((NUM_HEADS, n1, n1), jnp.float32))
    z_col = jnp.zeros((NUM_HEADS, N_NODE, 1), jnp.float32)
    z_row = jnp.zeros((NUM_HEADS, 1, n1), jnp.float32)
    padded = jnp.concatenate(
        [z_row, jnp.concatenate([z_col, inner], axis=2)], axis=1)
    o_ref[0] = ab2[None] + tb + padded


def _tc_asm(attn_bias, rt4, gtvd_w):
    n1 = N_NODE + 1
    return pl.pallas_call(
        _asm_body,
        grid=(N_GRAPH,),
        in_specs=[
            pl.BlockSpec((1, n1, n1), lambda g: (g, 0, 0)),
            pl.BlockSpec((NUM_HEADS, 1, N_NODE, N_NODE), lambda g: (0, g, 0, 0)),
            pl.BlockSpec((1, NUM_HEADS), lambda g: (0, 0)),
        ],
        out_specs=pl.BlockSpec((1, NUM_HEADS, n1, n1), lambda g: (g, 0, 0, 0)),
        out_shape=jax.ShapeDtypeStruct(
            (N_GRAPH, NUM_HEADS, n1, n1), jnp.float32),
    )(attn_bias, rt4, gtvd_w)


def kernel(attn_bias, node_attr, is_molecule, spatial_pos, edge_input,
           spatial_pos_w, gtvd_w, edge_enc_w, edge_dis_w):
    w = edge_dis_w.reshape(-1, EDGE_HIDDEN, NUM_HEADS)[:MULTI_HOP_MAX_DIST]
    eew_pad = jnp.zeros((_TROWS_P, EDGE_HIDDEN), jnp.float32)
    eew_pad = eew_pad.at[:_TBL_ROWS].set(edge_enc_w)
    t5 = _tc_pre(eew_pad, w)

    spw_pad = jnp.zeros((_TROWS_P, NUM_HEADS), jnp.float32)
    spw_pad = spw_pad.at[:spatial_pos_w.shape[0]].set(spatial_pos_w)
    all6 = jnp.concatenate([t5, spw_pad[None]], axis=0)
    lo = lax.bitcast_convert_type(
        all6[..., 0::2].astype(jnp.bfloat16), jnp.uint16).astype(jnp.uint32)
    hi = lax.bitcast_convert_type(
        all6[..., 1::2].astype(jnp.bfloat16), jnp.uint16).astype(jnp.uint32)
    packed = lax.bitcast_convert_type(lo | (hi << 16), jnp.int32)
    packed = jnp.pad(packed, ((0, 0), (0, 0), (0, 1)))
    packed = packed.reshape(6 * _TSIZE)

    idx_t = jnp.transpose(
        edge_input.reshape(_NPAIR, MULTI_HOP_MAX_DIST, 3), (1, 2, 0))
    idx_t = idx_t.reshape(MULTI_HOP_MAX_DIST * 3 * _NPAIR)
    spos_flat = spatial_pos.reshape(_NPAIR)

    rt = _sc_kernel(packed, idx_t, spos_flat)
    rt4 = rt.reshape(NUM_HEADS, N_GRAPH, N_NODE, N_NODE)
    return _tc_asm(attn_bias, rt4, gtvd_w)


# speedup vs baseline: 27.3454x; 1.0681x over previous
import jax
import jax.numpy as jnp
from jax import lax
from jax.experimental import pallas as pl
from jax.experimental.pallas import tpu as pltpu
from jax.experimental.pallas import tpu_sc as plsc

NUM_HEADS = 32
EDGE_HIDDEN = 32
MULTI_HOP_MAX_DIST = 5
NUM_EDGES = 1536
N_GRAPH = 16
N_NODE = 64

_NPAIR = N_GRAPH * N_NODE * N_NODE
_TBL_ROWS = NUM_EDGES + 1
_TROWS_P = 1544
_TSTRIDE = 17
_TSIZE = _TROWS_P * _TSTRIDE

_info = plsc.get_sparse_core_info()
_NC, _NS = _info.num_cores, _info.num_subcores
_NW = _NC * _NS
_CHUNK = _NPAIR // _NW
_GROUPS = _CHUNK // 16


def _tpre_body(e_ref, w_ref, o_ref):
    o_ref[0] = jnp.dot(e_ref[...], w_ref[0], preferred_element_type=jnp.float32)


def _tc_pre(edge_enc_w_pad, w):
    return pl.pallas_call(
        _tpre_body,
        grid=(MULTI_HOP_MAX_DIST,),
        in_specs=[
            pl.BlockSpec((_TROWS_P, EDGE_HIDDEN), lambda d: (0, 0)),
            pl.BlockSpec((1, EDGE_HIDDEN, NUM_HEADS), lambda d: (d, 0, 0)),
        ],
        out_specs=pl.BlockSpec((1, _TROWS_P, NUM_HEADS), lambda d: (d, 0, 0)),
        out_shape=jax.ShapeDtypeStruct(
            (MULTI_HOP_MAX_DIST, _TROWS_P, NUM_HEADS), jnp.float32),
    )(edge_enc_w_pad, w)


def _gather_pair(tbl, addr):
    w = plsc.load_gather(tbl, [addr])
    bf = plsc.bitcast(w, jnp.bfloat16)
    return plsc.unpack(bf, format=plsc.PackFormat.INTERLEAVED)


def _sc_body(tp_hbm, idx_hbm, spos_hbm, out_hbm, tbl_a, tbl_b, acc, idx6):
    wid = lax.axis_index("s") * _NC + lax.axis_index("c")
    base = wid * _CHUNK

    def accslot(g, c):
        return acc.at[c, pl.ds(pl.multiple_of(g * 16, 16), 16)]

    for rnd, (da, db) in enumerate(((0, 1), (2, 3))):
        pltpu.sync_copy(tp_hbm.at[pl.ds(da * _TSIZE, _TSIZE)], tbl_a)
        pltpu.sync_copy(tp_hbm.at[pl.ds(db * _TSIZE, _TSIZE)], tbl_b)
        for k in range(3):
            pltpu.sync_copy(
                idx_hbm.at[pl.ds((da * 3 + k) * _NPAIR + base, _CHUNK)],
                idx6.at[pl.ds(k * _CHUNK, _CHUNK)])
            pltpu.sync_copy(
                idx_hbm.at[pl.ds((db * 3 + k) * _NPAIR + base, _CHUNK)],
                idx6.at[pl.ds((3 + k) * _CHUNK, _CHUNK)])

        def eround(g, carry, first=(rnd == 0)):
            p0 = pl.multiple_of(g * 16, 16)
            ia = [idx6[pl.ds(k * _CHUNK + p0, 16)] * _TSTRIDE for k in range(3)]
            ib = [idx6[pl.ds((3 + k) * _CHUNK + p0, 16)] * _TSTRIDE
                  for k in range(3)]
            for cp in range(NUM_HEADS // 2):
                cc = jnp.full((16,), cp, jnp.int32)
                lo = None
                hi = None
                for k in range(3):
                    la, ha = _gather_pair(tbl_a, ia[k] + cc)
                    lb, hb = _gather_pair(tbl_b, ib[k] + cc)
                    lo = la + lb if lo is None else lo + la + lb
                    hi = ha + hb if hi is None else hi + ha + hb
                if first:
                    accslot(g, 2 * cp)[...] = lo
                    accslot(g, 2 * cp + 1)[...] = hi
                else:
                    plsc.addupdate(accslot(g, 2 * cp), lo)
                    plsc.addupdate(accslot(g, 2 * cp + 1), hi)
            return carry

        lax.fori_loop(0, _GROUPS, eround, 0)

    pltpu.sync_copy(tp_hbm.at[pl.ds(4 * _TSIZE, _TSIZE)], tbl_a)
    pltpu.sync_copy(tp_hbm.at[pl.ds(5 * _TSIZE, _TSIZE)], tbl_b)
    for k in range(3):
        pltpu.sync_copy(
            idx_hbm.at[pl.ds((4 * 3 + k) * _NPAIR + base, _CHUNK)],
            idx6.at[pl.ds(k * _CHUNK, _CHUNK)])
    pltpu.sync_copy(spos_hbm.at[pl.ds(base, _CHUNK)],
                    idx6.at[pl.ds(3 * _CHUNK, _CHUNK)])

    def sround(g, carry):
        p0 = pl.multiple_of(g * 16, 16)
        ia = [idx6[pl.ds(k * _CHUNK + p0, 16)] * _TSTRIDE for k in range(3)]
        sv = idx6[pl.ds(3 * _CHUNK + p0, 16)]
        sp = jnp.where(sv == 0, 1, sv)
        sp = jnp.where(sp > 1, sp - 1, sp)
        sp = jnp.minimum(sp, MULTI_HOP_MAX_DIST)
        recip = (1.0 / 3.0) / sp.astype(jnp.float32)
        svs = sv * _TSTRIDE
        for cp in range(NUM_HEADS // 2):
            cc = jnp.full((16,), cp, jnp.int32)
            lo = None
            hi = None
            for k in range(3):
                la, ha = _gather_pair(tbl_a, ia[k] + cc)
                lo = la if lo is None else lo + la
                hi = ha if hi is None else hi + ha
            slo, shi = _gather_pair(tbl_b, svs + cc)
            s_lo = accslot(g, 2 * cp)
            s_hi = accslot(g, 2 * cp + 1)
            s_lo[...] = (s_lo[...] + lo) * recip + slo
            s_hi[...] = (s_hi[...] + hi) * recip + shi
        return carry

    lax.fori_loop(0, _GROUPS, sround, 0)

    pltpu.sync_copy(acc, out_hbm.at[:, pl.ds(base, _CHUNK)])


_sc_kernel = pl.kernel(
    _sc_body,
    out_type=jax.ShapeDtypeStruct((NUM_HEADS, _NPAIR), jnp.float32),
    mesh=plsc.VectorSubcoreMesh(core_axis_name="c", subcore_axis_name="s"),
    compiler_params=pltpu.CompilerParams(needs_layout_passes=False),
    scratch_types=[
        pltpu.VMEM((_TSIZE,), jnp.int32),
        pltpu.VMEM((_TSIZE,), jnp.int32),
        pltpu.VMEM((NUM_HEADS, _CHUNK), jnp.float32),
        pltpu.VMEM((6 * _CHUNK,), jnp.int32),
    ],
)


def _asm_body(ab_ref, rt_ref, t_ref, o_ref):
    ab2 = ab_ref[0] * 2.0
    inner = rt_ref[...].reshape(NUM_HEADS, N_NODE, N_NODE)
    t = t_ref[0]
    n1 = N_NODE + 1
    ii = lax.broadcasted_iota(jnp.int32, (NUM_HEADS, n1, n1), 1)
    jj = lax.broadcasted_iota(jnp.int32, (NUM_HEADS, n1, n1), 2)
    border = (ii == 0) | (jj == 0)
    tb = jnp.where(border,
                   jnp.broadcast_to(t[:, None, None], (NUM_HEADS, n1, n1)),
                   jnp.zeros((NUM_HEADS, n1, n1), jnp.float32))
    z_col = jnp.zeros((NUM_HEADS, N_NODE, 1), jnp.float32)
    z_row = jnp.zeros((NUM_HEADS, 1, n1), jnp.float32)
    padded = jnp.concatenate(
        [z_row, jnp.concatenate([z_col, inner], axis=2)], axis=1)
    o_ref[0] = ab2[None] + tb + padded


def _tc_asm(attn_bias, rt, gtvd_w):
    n1 = N_NODE + 1
    return pl.pallas_call(
        _asm_body,
        grid=(N_GRAPH,),
        in_specs=[
            pl.BlockSpec((1, n1, n1), lambda g: (g, 0, 0)),
            pl.BlockSpec((NUM_HEADS, N_NODE * N_NODE), lambda g: (0, g)),
            pl.BlockSpec((1, NUM_HEADS), lambda g: (0, 0)),
        ],
        out_specs=pl.BlockSpec((1, NUM_HEADS, n1, n1), lambda g: (g, 0, 0, 0)),
        out_shape=jax.ShapeDtypeStruct(
            (N_GRAPH, NUM_HEADS, n1, n1), jnp.float32),
    )(attn_bias, rt, gtvd_w)


def kernel(attn_bias, node_attr, is_molecule, spatial_pos, edge_input,
           spatial_pos_w, gtvd_w, edge_enc_w, edge_dis_w):
    w = edge_dis_w.reshape(-1, EDGE_HIDDEN, NUM_HEADS)[:MULTI_HOP_MAX_DIST]
    eew_pad = jnp.zeros((_TROWS_P, EDGE_HIDDEN), jnp.float32)
    eew_pad = eew_pad.at[:_TBL_ROWS].set(edge_enc_w)
    t5 = _tc_pre(eew_pad, w)

    spw_pad = jnp.zeros((_TROWS_P, NUM_HEADS), jnp.float32)
    spw_pad = spw_pad.at[:spatial_pos_w.shape[0]].set(spatial_pos_w)
    all6 = jnp.concatenate([t5, spw_pad[None]], axis=0)
    lo = lax.bitcast_convert_type(
        all6[..., 0::2].astype(jnp.bfloat16), jnp.uint16).astype(jnp.uint32)
    hi = lax.bitcast_convert_type(
        all6[..., 1::2].astype(jnp.bfloat16), jnp.uint16).astype(jnp.uint32)
    packed = lax.bitcast_convert_type(lo | (hi << 16), jnp.int32)
    packed = jnp.pad(packed, ((0, 0), (0, 0), (0, 1)))
    packed = packed.reshape(6 * _TSIZE)

    idx_t = jnp.transpose(
        edge_input.reshape(_NPAIR, MULTI_HOP_MAX_DIST, 3), (1, 2, 0))
    idx_t = idx_t.reshape(MULTI_HOP_MAX_DIST * 3 * _NPAIR)
    spos_flat = spatial_pos.reshape(_NPAIR)

    rt = _sc_kernel(packed, idx_t, spos_flat)
    return _tc_asm(attn_bias, rt, gtvd_w)
